# baseline (speedup 1.0000x reference)
import functools

import jax
import jax.numpy as jnp
from jax import lax
from jax.experimental import pallas as pl
from jax.experimental.pallas import tpu as pltpu
from jax.experimental.pallas import tpu_sc as plsc

_VOCAB = 1000
_TILE_ROWS = 32


@functools.lru_cache(maxsize=None)
def _make_sc_onehot(n_rows: int, vocab: int):
    info = plsc.get_sparse_core_info()
    num_cores, num_subcores, lanes = (
        info.num_cores, info.num_subcores, info.num_lanes)
    rows_per_core = n_rows // num_cores
    assert rows_per_core * num_cores == n_rows

    rows_per_tile_chunk = _TILE_ROWS
    chunk_rows = rows_per_tile_chunk * num_subcores
    n_chunks = rows_per_core // chunk_rows
    assert n_chunks * chunk_rows == rows_per_core
    half_words = chunk_rows * vocab
    slot_words = rows_per_tile_chunk * vocab

    mesh = plsc.VectorSubcoreMesh(core_axis_name="c", subcore_axis_name="s")

    @functools.partial(
        pl.kernel,
        mesh=mesh,
        out_type=jax.ShapeDtypeStruct(
            (num_cores * n_chunks, half_words), jnp.float32),
        scratch_types=[
            pltpu.VMEM_SHARED((half_words,), jnp.float32),
            pltpu.VMEM_SHARED((half_words,), jnp.float32),
            pltpu.VMEM((slot_words,), jnp.float32),
            pltpu.VMEM((rows_per_tile_chunk,), jnp.int32),
            pltpu.VMEM((rows_per_tile_chunk,), jnp.int32),
            pltpu.VMEM((rows_per_tile_chunk,), jnp.int32),
            pltpu.VMEM((rows_per_tile_chunk,), jnp.float32),
            pltpu.VMEM((rows_per_tile_chunk,), jnp.float32),
            pltpu.SemaphoreType.DMA,
            pltpu.SemaphoreType.DMA,
        ],
        compiler_params=pltpu.CompilerParams(needs_layout_passes=False),
    )
    def onehot(idx_hbm, out_hbm, half_a, half_b, zbuf, xbuf,
               idx_a, idx_b, ones_v, zeros_v, sem_a, sem_b):
        halves = (half_a, half_b)
        idx_bufs = (idx_a, idx_b)
        sems = (sem_a, sem_b)
        cid = lax.axis_index("c")
        sid = lax.axis_index("s")

        zeros16 = jnp.zeros((lanes,), jnp.float32)
        ones16 = jnp.ones((lanes,), jnp.float32)
        lane = lax.iota(jnp.int32, lanes)

        for j in range(rows_per_tile_chunk // lanes):
            ones_v[pl.ds(j * lanes, lanes)] = ones16
            zeros_v[pl.ds(j * lanes, lanes)] = zeros16

        def fill_body(i, c):
            zbuf[pl.ds(i * lanes, lanes)] = zeros16
            return c

        lax.fori_loop(0, slot_words // lanes, fill_body, 0)

        slot0 = pl.multiple_of(sid * slot_words, 8)
        pltpu.sync_copy(zbuf, half_a.at[pl.ds(slot0, slot_words)])
        pltpu.sync_copy(zbuf, half_b.at[pl.ds(slot0, slot_words)])
        plsc.subcore_barrier()

        handles = {}
        for c in range(n_chunks):
            half = halves[c % 2]
            idxb = idx_bufs[c % 2]
            if c >= 2:
                @pl.when(sid == 0)
                def _(h=handles[c - 2]):
                    h.wait()
                plsc.subcore_barrier()
                pltpu.sync_copy(zeros_v, half.at[idxb])
            idx_row = cid * (n_chunks * num_subcores) + c * num_subcores + sid
            pltpu.sync_copy(idx_hbm.at[idx_row], xbuf)
            for j in range(rows_per_tile_chunk // lanes):
                xv = xbuf[pl.ds(j * lanes, lanes)]
                pos = (sid * rows_per_tile_chunk + j * lanes + lane) * vocab + xv
                idxb[pl.ds(j * lanes, lanes)] = pos
            pltpu.sync_copy(ones_v, half.at[idxb])
            plsc.subcore_barrier()

            handles[c] = pltpu.make_async_copy(
                half, out_hbm.at[cid * n_chunks + c], sems[c % 2])

            @pl.when(sid == 0)
            def _(h=handles[c]):
                h.start()

        @pl.when(sid == 0)
        def _():
            if n_chunks >= 2:
                handles[n_chunks - 2].wait()
            handles[n_chunks - 1].wait()

    return onehot


def kernel(x):
    n0, n1 = x.shape
    idx = x.reshape(-1, _TILE_ROWS).astype(jnp.int32)
    out = _make_sc_onehot(n0 * n1, _VOCAB)(idx)
    return out.reshape(n0, n1, _VOCAB)


# speedup vs baseline: 1.3436x; 1.3436x over previous
import functools

import jax
import jax.numpy as jnp
from jax import lax
from jax.experimental import pallas as pl
from jax.experimental.pallas import tpu as pltpu
from jax.experimental.pallas import tpu_sc as plsc

_VOCAB = 1000
_TILE_ROWS = 32


@functools.lru_cache(maxsize=None)
def _make_sc_onehot(n_rows: int, vocab: int):
    info = plsc.get_sparse_core_info()
    num_cores, num_subcores, lanes = (
        info.num_cores, info.num_subcores, info.num_lanes)
    rows_per_core = n_rows // num_cores
    assert rows_per_core * num_cores == n_rows

    rows_per_tile_chunk = _TILE_ROWS
    chunk_rows = rows_per_tile_chunk * num_subcores
    n_chunks = rows_per_core // chunk_rows
    assert n_chunks * chunk_rows == rows_per_core
    half_words = chunk_rows * vocab
    slot_words = rows_per_tile_chunk * vocab

    mesh = plsc.VectorSubcoreMesh(core_axis_name="c", subcore_axis_name="s")

    @functools.partial(
        pl.kernel,
        mesh=mesh,
        out_type=jax.ShapeDtypeStruct((n_rows * vocab,), jnp.float32),
        scratch_types=[
            pltpu.VMEM_SHARED((half_words,), jnp.float32),
            pltpu.VMEM_SHARED((half_words,), jnp.float32),
            pltpu.VMEM((slot_words,), jnp.float32),
            pltpu.VMEM((rows_per_tile_chunk,), jnp.int32),
            pltpu.VMEM((rows_per_tile_chunk,), jnp.int32),
            pltpu.VMEM((rows_per_tile_chunk,), jnp.int32),
            pltpu.VMEM((rows_per_tile_chunk,), jnp.float32),
            pltpu.VMEM((rows_per_tile_chunk,), jnp.float32),
            pltpu.SemaphoreType.DMA,
            pltpu.SemaphoreType.DMA,
        ],
        compiler_params=pltpu.CompilerParams(needs_layout_passes=False),
    )
    def onehot(idx_hbm, out_hbm, half_a, half_b, zbuf, xbuf,
               idx_a, idx_b, ones_v, zeros_v, sem_a, sem_b):
        halves = (half_a, half_b)
        idx_bufs = (idx_a, idx_b)
        sems = (sem_a, sem_b)
        cid = lax.axis_index("c")
        sid = lax.axis_index("s")

        zeros16 = jnp.zeros((lanes,), jnp.float32)
        ones16 = jnp.ones((lanes,), jnp.float32)
        lane = lax.iota(jnp.int32, lanes)

        for j in range(rows_per_tile_chunk // lanes):
            ones_v[pl.ds(j * lanes, lanes)] = ones16
            zeros_v[pl.ds(j * lanes, lanes)] = zeros16

        def fill_body(i, c):
            zbuf[pl.ds(i * lanes, lanes)] = zeros16
            return c

        lax.fori_loop(0, slot_words // lanes, fill_body, 0)

        slot0 = pl.multiple_of(sid * slot_words, 8)
        pltpu.sync_copy(zbuf, half_a.at[pl.ds(slot0, slot_words)])
        pltpu.sync_copy(zbuf, half_b.at[pl.ds(slot0, slot_words)])
        plsc.subcore_barrier()

        handles = {}
        for c in range(n_chunks):
            half = halves[c % 2]
            idxb = idx_bufs[c % 2]
            if c >= 2:
                @pl.when(sid == 0)
                def _(h=handles[c - 2]):
                    h.wait()
                plsc.subcore_barrier()
                pltpu.sync_copy(zeros_v, half.at[idxb])
            idx_row = cid * (n_chunks * num_subcores) + c * num_subcores + sid
            pltpu.sync_copy(idx_hbm.at[idx_row], xbuf)
            for j in range(rows_per_tile_chunk // lanes):
                xv = xbuf[pl.ds(j * lanes, lanes)]
                pos = (sid * rows_per_tile_chunk + j * lanes + lane) * vocab + xv
                idxb[pl.ds(j * lanes, lanes)] = pos
            pltpu.sync_copy(ones_v, half.at[idxb])
            plsc.subcore_barrier()

            handles[c] = pltpu.make_async_copy(
                half,
                out_hbm.at[pl.ds((cid * n_chunks + c) * half_words,
                                 half_words)],
                sems[c % 2])

            @pl.when(sid == 0)
            def _(h=handles[c]):
                h.start()

        @pl.when(sid == 0)
        def _():
            if n_chunks >= 2:
                handles[n_chunks - 2].wait()
            handles[n_chunks - 1].wait()

    return onehot


def kernel(x):
    n0, n1 = x.shape
    idx = x.reshape(-1, _TILE_ROWS).astype(jnp.int32)
    out = _make_sc_onehot(n0 * n1, _VOCAB)(idx)
    return out.reshape(n0, n1, _VOCAB)


# speedup vs baseline: 5.0244x; 3.7396x over previous
import functools

import jax
import jax.numpy as jnp
from jax import lax
from jax.experimental import pallas as pl
from jax.experimental.pallas import tpu as pltpu
from jax.experimental.pallas import tpu_sc as plsc

_VOCAB = 1000


@functools.lru_cache(maxsize=None)
def _make_sc_onehot(n_batch: int, n_pos: int, vocab: int):
    info = plsc.get_sparse_core_info()
    num_cores, num_subcores, lanes = (
        info.num_cores, info.num_subcores, info.num_lanes)
    planes_per_core = n_pos // num_cores
    assert planes_per_core * num_cores == n_pos
    assert vocab % 8 == 0 and n_batch % 128 == 0
    plane_words = vocab * n_batch
    rows_per_tile = n_batch // num_subcores
    n_vec = rows_per_tile // lanes

    tile_row_words = 8 * n_batch
    n_tile_rows = vocab // 8
    split = n_tile_rows // 2
    chunk_words = (split * tile_row_words,
                   (n_tile_rows - split) * tile_row_words)
    trash_base = max(chunk_words)
    align = num_subcores * lanes
    buf_words = ((trash_base + rows_per_tile + align - 1) // align) * align
    slot_words = buf_words // num_subcores

    mesh = plsc.VectorSubcoreMesh(core_axis_name="c", subcore_axis_name="s")

    @functools.partial(
        pl.kernel,
        mesh=mesh,
        out_type=jax.ShapeDtypeStruct((n_pos * plane_words,), jnp.float32),
        scratch_types=[
            pltpu.VMEM_SHARED((buf_words,), jnp.float32),
            pltpu.VMEM_SHARED((buf_words,), jnp.float32),
            pltpu.VMEM((slot_words,), jnp.float32),
            pltpu.VMEM((rows_per_tile,), jnp.int32),
            pltpu.VMEM((rows_per_tile,), jnp.int32),
            pltpu.VMEM((rows_per_tile,), jnp.int32),
            pltpu.VMEM((rows_per_tile,), jnp.float32),
            pltpu.VMEM((rows_per_tile,), jnp.float32),
            pltpu.SemaphoreType.DMA,
            pltpu.SemaphoreType.DMA,
        ],
        compiler_params=pltpu.CompilerParams(needs_layout_passes=False),
    )
    def onehot(xt_hbm, out_hbm, buf_a, buf_b, zbuf, xbuf,
               idx_a, idx_b, ones_v, zeros_v, sem_a, sem_b):
        bufs = (buf_a, buf_b)
        idx_bufs = (idx_a, idx_b)
        sems = (sem_a, sem_b)
        cid = lax.axis_index("c")
        sid = lax.axis_index("s")

        zeros16 = jnp.zeros((lanes,), jnp.float32)
        ones16 = jnp.ones((lanes,), jnp.float32)
        lane = lax.iota(jnp.int32, lanes)

        for j in range(n_vec):
            ones_v[pl.ds(j * lanes, lanes)] = ones16
            zeros_v[pl.ds(j * lanes, lanes)] = zeros16

        def fill_body(i, c):
            zbuf[pl.ds(i * lanes, lanes)] = zeros16
            return c

        lax.fori_loop(0, slot_words // lanes, fill_body, 0)

        slot0 = sid * slot_words
        pltpu.sync_copy(zbuf, buf_a.at[pl.ds(slot0, slot_words)])
        pltpu.sync_copy(zbuf, buf_b.at[pl.ds(slot0, slot_words)])
        plsc.subcore_barrier()

        n_chunks = 2 * planes_per_core
        handles = {}
        for k in range(n_chunks):
            plane, part = k // 2, k % 2
            buf = bufs[k % 2]
            idxb = idx_bufs[k % 2]
            if k >= 2:
                @pl.when(sid == 0)
                def _(h=handles[k - 2]):
                    h.wait()
                plsc.subcore_barrier()
                pltpu.sync_copy(zeros_v, buf.at[idxb])
            if part == 0:
                xt_row = (cid * planes_per_core + plane) * num_subcores + sid
                pltpu.sync_copy(xt_hbm.at[xt_row], xbuf)
            for j in range(n_vec):
                xv = xbuf[pl.ds(j * lanes, lanes)]
                b = sid * rows_per_tile + j * lanes + lane
                tv = xv >> 3
                rest = ((xv & 7) << 7) + ((b >> 7) << 10) + (b & 127)
                if part == 0:
                    pos = jnp.where(tv < split, tv * tile_row_words + rest,
                                    trash_base + j * lanes + lane)
                else:
                    pos = jnp.where(tv >= split,
                                    (tv - split) * tile_row_words + rest,
                                    trash_base + j * lanes + lane)
                idxb[pl.ds(j * lanes, lanes)] = pos
            pltpu.sync_copy(ones_v, buf.at[idxb])
            plsc.subcore_barrier()

            out_off = (cid * (planes_per_core * plane_words)
                       + plane * plane_words + part * chunk_words[0])
            handles[k] = pltpu.make_async_copy(
                buf.at[pl.ds(0, chunk_words[part])],
                out_hbm.at[pl.ds(out_off, chunk_words[part])],
                sems[k % 2])

            @pl.when(sid == 0)
            def _(h=handles[k]):
                h.start()

        @pl.when(sid == 0)
        def _():
            handles[n_chunks - 2].wait()
            handles[n_chunks - 1].wait()

    return onehot


def kernel(x):
    n0, n1 = x.shape
    xt = x.astype(jnp.int32).T.reshape(n1 * 16, n0 // 16)
    flat = _make_sc_onehot(n0, n1, _VOCAB)(xt)
    y = flat.reshape(n1, _VOCAB // 8, n0 // 128, 8, 128)
    return jnp.transpose(y, (2, 4, 0, 1, 3)).reshape(n0, n1, _VOCAB)


# speedup vs baseline: 5.6920x; 1.1329x over previous
import functools

import jax
import jax.numpy as jnp
from jax import lax
from jax.experimental import pallas as pl
from jax.experimental.pallas import tpu as pltpu
from jax.experimental.pallas import tpu_sc as plsc

_VOCAB = 1000


@functools.lru_cache(maxsize=None)
def _make_sc_onehot(n_batch: int, n_pos: int, vocab: int):
    info = plsc.get_sparse_core_info()
    num_cores, num_subcores, lanes = (
        info.num_cores, info.num_subcores, info.num_lanes)
    planes_per_core = n_pos // num_cores
    assert planes_per_core * num_cores == n_pos
    assert vocab % 8 == 0 and n_batch % 128 == 0
    plane_words = vocab * n_batch
    rows_per_tile = n_batch // num_subcores
    n_vec = rows_per_tile // lanes

    tile_row_words = 8 * n_batch
    n_tile_rows = vocab // 8
    split = n_tile_rows // 2
    chunk_words = (split * tile_row_words,
                   (n_tile_rows - split) * tile_row_words)
    trash_base = max(chunk_words)
    align = num_subcores * lanes
    buf_words = ((trash_base + rows_per_tile + align - 1) // align) * align
    slot_words = buf_words // num_subcores

    mesh = plsc.VectorSubcoreMesh(core_axis_name="c", subcore_axis_name="s")

    @functools.partial(
        pl.kernel,
        mesh=mesh,
        out_type=jax.ShapeDtypeStruct((n_pos * plane_words,), jnp.float32),
        scratch_types=[
            pltpu.VMEM_SHARED((buf_words,), jnp.float32),
            pltpu.VMEM_SHARED((buf_words,), jnp.float32),
            pltpu.VMEM((slot_words,), jnp.float32),
            pltpu.VMEM((planes_per_core * rows_per_tile,), jnp.int32),
            pltpu.VMEM((2 * rows_per_tile,), jnp.int32),
            pltpu.VMEM((2 * rows_per_tile,), jnp.int32),
            pltpu.VMEM((2 * rows_per_tile,), jnp.float32),
            pltpu.VMEM((2 * rows_per_tile,), jnp.float32),
            pltpu.SemaphoreType.DMA,
            pltpu.SemaphoreType.DMA,
        ],
        compiler_params=pltpu.CompilerParams(needs_layout_passes=False),
    )
    def onehot(xt_hbm, out_hbm, buf_a, buf_b, zbuf, xv_all,
               idx_a, idx_b, val_a, val_b, sem_a, sem_b):
        bufs = (buf_a, buf_b)
        idx_bufs = (idx_a, idx_b)
        val_bufs = (val_a, val_b)
        sems = (sem_a, sem_b)
        cid = lax.axis_index("c")
        sid = lax.axis_index("s")

        zeros16 = jnp.zeros((lanes,), jnp.float32)
        ones16 = jnp.ones((lanes,), jnp.float32)
        lane = lax.iota(jnp.int32, lanes)

        for j in range(n_vec):
            trash16 = trash_base + j * lanes + lane
            for vb, ib in ((val_a, idx_a), (val_b, idx_b)):
                vb[pl.ds(j * lanes, lanes)] = zeros16
                vb[pl.ds(rows_per_tile + j * lanes, lanes)] = ones16
                ib[pl.ds(rows_per_tile + j * lanes, lanes)] = trash16

        pltpu.sync_copy(
            xt_hbm.at[pl.ds((cid * num_subcores + sid)
                            * (planes_per_core * rows_per_tile),
                            planes_per_core * rows_per_tile)],
            xv_all)

        def fill_body(i, c):
            zbuf[pl.ds(i * lanes, lanes)] = zeros16
            return c

        lax.fori_loop(0, slot_words // lanes, fill_body, 0)

        slot0 = sid * slot_words
        pltpu.sync_copy(zbuf, buf_a.at[pl.ds(slot0, slot_words)])
        pltpu.sync_copy(zbuf, buf_b.at[pl.ds(slot0, slot_words)])
        plsc.subcore_barrier()

        n_chunks = 2 * planes_per_core
        handles = {}
        for k in range(n_chunks):
            plane, part = k // 2, k % 2
            buf = bufs[k % 2]
            idxb = idx_bufs[k % 2]
            valb = val_bufs[k % 2]
            if k >= 2:
                @pl.when(sid == 0)
                def _(h=handles[k - 2]):
                    h.wait()
                plsc.subcore_barrier()
            for j in range(n_vec):
                xv = xv_all[pl.ds(plane * rows_per_tile + j * lanes, lanes)]
                b = sid * rows_per_tile + j * lanes + lane
                tv = xv >> 3
                rest = ((xv & 7) << 7) + ((b >> 7) << 10) + (b & 127)
                if part == 0:
                    pos = jnp.where(tv < split, tv * tile_row_words + rest,
                                    trash_base + j * lanes + lane)
                else:
                    pos = jnp.where(tv >= split,
                                    (tv - split) * tile_row_words + rest,
                                    trash_base + j * lanes + lane)
                stale = idxb[pl.ds(rows_per_tile + j * lanes, lanes)]
                idxb[pl.ds(j * lanes, lanes)] = stale
                valb[pl.ds(j * lanes, lanes)] = jnp.where(
                    stale == pos, ones16, zeros16)
                idxb[pl.ds(rows_per_tile + j * lanes, lanes)] = pos
            pltpu.sync_copy(valb, buf.at[idxb])
            plsc.subcore_barrier()

            out_off = (cid * (planes_per_core * plane_words)
                       + plane * plane_words + part * chunk_words[0])
            handles[k] = pltpu.make_async_copy(
                buf.at[pl.ds(0, chunk_words[part])],
                out_hbm.at[pl.ds(out_off, chunk_words[part])],
                sems[k % 2])

            @pl.when(sid == 0)
            def _(h=handles[k]):
                h.start()

        @pl.when(sid == 0)
        def _():
            handles[n_chunks - 2].wait()
            handles[n_chunks - 1].wait()

    return onehot


def kernel(x):
    n0, n1 = x.shape
    xt = (x.astype(jnp.int32).T
          .reshape(2, n1 // 2, 16, n0 // 16)
          .transpose(0, 2, 1, 3)
          .reshape(-1))
    flat = _make_sc_onehot(n0, n1, _VOCAB)(xt)
    y = flat.reshape(n1, _VOCAB // 8, n0 // 128, 8, 128)
    return jnp.transpose(y, (2, 4, 0, 1, 3)).reshape(n0, n1, _VOCAB)


# speedup vs baseline: 5.8092x; 1.0206x over previous
import functools

import jax
import jax.numpy as jnp
from jax import lax
from jax.experimental import pallas as pl
from jax.experimental.pallas import tpu as pltpu
from jax.experimental.pallas import tpu_sc as plsc

_VOCAB = 1000


@functools.lru_cache(maxsize=None)
def _make_sc_onehot(n_batch: int, n_pos: int, vocab: int):
    info = plsc.get_sparse_core_info()
    num_cores, num_subcores, lanes = (
        info.num_cores, info.num_subcores, info.num_lanes)
    planes_per_core = n_pos // num_cores
    assert planes_per_core * num_cores == n_pos
    assert vocab % 8 == 0 and n_batch % 128 == 0
    plane_words = vocab * n_batch
    rows_per_tile = n_batch // num_subcores
    n_vec = rows_per_tile // lanes

    tile_row_words = 8 * n_batch
    n_tile_rows = vocab // 8
    split = n_tile_rows // 2
    chunk_words = (split * tile_row_words,
                   (n_tile_rows - split) * tile_row_words)
    trash_base = max(chunk_words)
    align = num_subcores * lanes
    buf_words = ((trash_base + rows_per_tile + align - 1) // align) * align
    slot_words = buf_words // num_subcores

    mesh = plsc.VectorSubcoreMesh(core_axis_name="c", subcore_axis_name="s")

    @functools.partial(
        pl.kernel,
        mesh=mesh,
        out_type=jax.ShapeDtypeStruct((n_pos * plane_words,), jnp.float32),
        scratch_types=[
            pltpu.VMEM_SHARED((buf_words,), jnp.float32),
            pltpu.VMEM_SHARED((buf_words,), jnp.float32),
            pltpu.VMEM((slot_words,), jnp.float32),
            pltpu.VMEM((planes_per_core * rows_per_tile,), jnp.int32),
            pltpu.VMEM((2 * rows_per_tile,), jnp.int32),
            pltpu.VMEM((2 * rows_per_tile,), jnp.int32),
            pltpu.VMEM((2 * rows_per_tile,), jnp.float32),
            pltpu.VMEM((2 * rows_per_tile,), jnp.float32),
            pltpu.SemaphoreType.DMA,
            pltpu.SemaphoreType.DMA,
        ],
        compiler_params=pltpu.CompilerParams(needs_layout_passes=False),
    )
    def onehot(xt_hbm, out_hbm, buf_a, buf_b, zbuf, xv_all,
               idx_a, idx_b, val_a, val_b, sem_a, sem_b):
        bufs = (buf_a, buf_b)
        idx_bufs = (idx_a, idx_b)
        val_bufs = (val_a, val_b)
        sems = (sem_a, sem_b)
        cid = lax.axis_index("c")
        sid = lax.axis_index("s")

        zeros16 = jnp.zeros((lanes,), jnp.float32)
        ones16 = jnp.ones((lanes,), jnp.float32)
        lane = lax.iota(jnp.int32, lanes)

        for j in range(n_vec):
            trash16 = trash_base + j * lanes + lane
            for vb, ib in ((val_a, idx_a), (val_b, idx_b)):
                vb[pl.ds(j * lanes, lanes)] = zeros16
                vb[pl.ds(rows_per_tile + j * lanes, lanes)] = ones16
                ib[pl.ds(rows_per_tile + j * lanes, lanes)] = trash16

        pltpu.sync_copy(
            xt_hbm.at[pl.ds((cid * num_subcores + sid)
                            * (planes_per_core * rows_per_tile),
                            planes_per_core * rows_per_tile)],
            xv_all)

        def fill_body(i, c):
            zbuf[pl.ds(i * lanes, lanes)] = zeros16
            return c

        lax.fori_loop(0, slot_words // lanes, fill_body, 0)

        slot0 = sid * slot_words
        pltpu.sync_copy(zbuf, buf_a.at[pl.ds(slot0, slot_words)])
        plsc.subcore_barrier()

        n_chunks = 2 * planes_per_core
        handles = {}
        for k in range(n_chunks):
            plane, part = k // 2, k % 2
            buf = bufs[k % 2]
            idxb = idx_bufs[k % 2]
            valb = val_bufs[k % 2]
            if k >= 2:
                @pl.when(sid == 0)
                def _(h=handles[k - 2]):
                    h.wait()
                plsc.subcore_barrier()
            for j in range(n_vec):
                xv = xv_all[pl.ds(plane * rows_per_tile + j * lanes, lanes)]
                b = sid * rows_per_tile + j * lanes + lane
                tv = xv >> 3
                rest = ((xv & 7) << 7) + ((b >> 7) << 10) + (b & 127)
                if part == 0:
                    pos = jnp.where(tv < split, tv * tile_row_words + rest,
                                    trash_base + j * lanes + lane)
                else:
                    pos = jnp.where(tv >= split,
                                    (tv - split) * tile_row_words + rest,
                                    trash_base + j * lanes + lane)
                stale = idxb[pl.ds(rows_per_tile + j * lanes, lanes)]
                idxb[pl.ds(j * lanes, lanes)] = stale
                valb[pl.ds(j * lanes, lanes)] = jnp.where(
                    stale == pos, ones16, zeros16)
                idxb[pl.ds(rows_per_tile + j * lanes, lanes)] = pos
            pltpu.sync_copy(valb, buf.at[idxb])
            plsc.subcore_barrier()

            out_off = (cid * (planes_per_core * plane_words)
                       + plane * plane_words + part * chunk_words[0])
            handles[k] = pltpu.make_async_copy(
                buf.at[pl.ds(0, chunk_words[part])],
                out_hbm.at[pl.ds(out_off, chunk_words[part])],
                sems[k % 2])

            @pl.when(sid == 0)
            def _(h=handles[k]):
                h.start()

            if k == 0:
                pltpu.sync_copy(zbuf, buf_b.at[pl.ds(slot0, slot_words)])
                plsc.subcore_barrier()

        @pl.when(sid == 0)
        def _():
            handles[n_chunks - 2].wait()
            handles[n_chunks - 1].wait()

    return onehot


def kernel(x):
    n0, n1 = x.shape
    xt = (x.astype(jnp.int32).T
          .reshape(2, n1 // 2, 16, n0 // 16)
          .transpose(0, 2, 1, 3)
          .reshape(-1))
    flat = _make_sc_onehot(n0, n1, _VOCAB)(xt)
    y = flat.reshape(n1, _VOCAB // 8, n0 // 128, 8, 128)
    return jnp.transpose(y, (2, 4, 0, 1, 3)).reshape(n0, n1, _VOCAB)
